# baseline (speedup 1.0000x reference)
import functools

import jax
import jax.numpy as jnp
from jax import lax
from jax.experimental import pallas as pl
from jax.experimental.pallas import tpu as pltpu
from jax.experimental.pallas import tpu_sc as plsc

_SIZE = 100000
_CATS = 100
_DIM = 64
_E = 1024

_BV = 2048


@functools.partial(jax.jit, static_argnums=(2, 3))
def _sc_gather(table, idx, B, D):
    info = plsc.get_sparse_core_info()
    NW = info.num_cores * info.num_subcores
    b_per_w = B // NW
    mesh = plsc.VectorSubcoreMesh(core_axis_name="c", subcore_axis_name="s")

    @functools.partial(
        pl.kernel,
        mesh=mesh,
        out_type=jax.ShapeDtypeStruct((B, D), jnp.float32),
        scratch_types=[
            pltpu.VMEM((b_per_w,), jnp.int32),
            pltpu.VMEM((b_per_w, D), jnp.float32),
            pltpu.SemaphoreType.DMA,
        ],
        compiler_params=pltpu.CompilerParams(use_tc_tiling_on_sc=False),
    )
    def k(table_hbm, idx_hbm, out_hbm, idx_v, rows_v, sem):
        wid = lax.axis_index("s") * info.num_cores + lax.axis_index("c")
        base = wid * b_per_w
        pltpu.sync_copy(idx_hbm.at[pl.ds(base, b_per_w)], idx_v)
        pltpu.async_copy(table_hbm.at[idx_v], rows_v, sem).wait()
        pltpu.sync_copy(rows_v, out_hbm.at[pl.ds(base, b_per_w)])

    return k(table, idx)


def _head_body(phiw_ref, phic_ref, wc_ref, bct_ref, gt_ref,
               priort_ref, zt_ref, nd_ref):
    phiw = phiw_ref[...]
    wc = wc_ref[...]
    bct = bct_ref[...]
    cw = phiw * phic_ref[...]
    logits_t = lax.dot_general(
        wc, cw, (((1,), (1,)), ((), ())),
        preferred_element_type=jnp.float32) + bct
    yt = jax.nn.softmax(logits_t + gt_ref[...], axis=0)
    rows = lax.broadcasted_iota(jnp.int32, (_CATS, _E), 0)
    ymax = jnp.max(yt, axis=0, keepdims=True)
    cand = jnp.where(yt >= ymax, rows, jnp.int32(2**30))
    first = jnp.min(cand, axis=0, keepdims=True)
    zt = (rows == first).astype(jnp.float32)
    zt_ref[...] = zt
    pl_t = lax.dot_general(
        wc, phiw, (((1,), (1,)), ((), ())),
        preferred_element_type=jnp.float32) + bct
    priort_ref[...] = jax.nn.softmax(pl_t, axis=0)
    ndv = lax.dot_general(
        zt, wc, (((0,), (0,)), ((), ())),
        preferred_element_type=jnp.float32)
    nd_ref[...] = jnp.concatenate(
        [ndv, jnp.ones((_E, 1), jnp.float32)], axis=1)


def _dec_body(wdt_ref, bdr_ref, nd_ref, out_ref):
    lhs = jnp.concatenate([wdt_ref[...], bdr_ref[...]], axis=0)
    out_ref[...] = lax.dot_general(
        lhs, nd_ref[...], (((0,), (1,)), ((), ())),
        preferred_element_type=jnp.float32)


def kernel(w, c, edge_index, node_emb, W_comm, b_comm, W_dec, b_dec):
    del edge_index
    idx_all = jnp.concatenate([w, c]).astype(jnp.int32)
    phi = _sc_gather(node_emb, idx_all, 2 * _E, _DIM)
    phi_w, phi_c = phi[:_E], phi[_E:]

    gt = jax.random.gumbel(jax.random.key(42), (_E, _CATS), jnp.float32).T
    bct = b_comm.reshape(_CATS, 1)
    prior_t, z_t, nd = pl.pallas_call(
        _head_body,
        out_shape=(
            jax.ShapeDtypeStruct((_CATS, _E), jnp.float32),
            jax.ShapeDtypeStruct((_CATS, _E), jnp.float32),
            jax.ShapeDtypeStruct((_E, _DIM + 1), jnp.float32),
        ),
    )(phi_w, phi_c, W_comm, bct, gt)

    bdr = b_dec.reshape(1, _SIZE)
    nb = pl.cdiv(_SIZE, _BV)
    recon_t = pl.pallas_call(
        _dec_body,
        grid=(nb,),
        in_specs=[
            pl.BlockSpec((_DIM, _BV), lambda i: (0, i)),
            pl.BlockSpec((1, _BV), lambda i: (0, i)),
            pl.BlockSpec((_E, _DIM + 1), lambda i: (0, 0)),
        ],
        out_specs=pl.BlockSpec((_BV, _E), lambda i: (i, 0)),
        out_shape=jax.ShapeDtypeStruct((_SIZE, _E), jnp.float32),
        compiler_params=pltpu.CompilerParams(
            dimension_semantics=("parallel",)),
    )(W_dec.T, bdr, nd)

    return (prior_t.T, recon_t.T, z_t.T)


# speedup vs baseline: 1.0077x; 1.0077x over previous
import functools

import jax
import jax.numpy as jnp
from jax import lax
from jax.experimental import pallas as pl
from jax.experimental.pallas import tpu as pltpu
from jax.experimental.pallas import tpu_sc as plsc

_SIZE = 100000
_CATS = 100
_DIM = 64
_E = 1024

_BV = 4096


@functools.partial(jax.jit, static_argnums=(2, 3))
def _sc_gather(table, idx, B, D):
    info = plsc.get_sparse_core_info()
    NW = info.num_cores * info.num_subcores
    b_per_w = B // NW
    mesh = plsc.VectorSubcoreMesh(core_axis_name="c", subcore_axis_name="s")

    @functools.partial(
        pl.kernel,
        mesh=mesh,
        out_type=jax.ShapeDtypeStruct((B, D), jnp.float32),
        scratch_types=[
            pltpu.VMEM((b_per_w,), jnp.int32),
            pltpu.VMEM((b_per_w, D), jnp.float32),
            pltpu.SemaphoreType.DMA,
        ],
        compiler_params=pltpu.CompilerParams(use_tc_tiling_on_sc=False),
    )
    def k(table_hbm, idx_hbm, out_hbm, idx_v, rows_v, sem):
        wid = lax.axis_index("s") * info.num_cores + lax.axis_index("c")
        base = wid * b_per_w
        pltpu.sync_copy(idx_hbm.at[pl.ds(base, b_per_w)], idx_v)
        pltpu.async_copy(table_hbm.at[idx_v], rows_v, sem).wait()
        pltpu.sync_copy(rows_v, out_hbm.at[pl.ds(base, b_per_w)])

    return k(table, idx)


def _head_body(phiw_ref, phic_ref, wc_ref, bct_ref, gt_ref,
               priort_ref, zt_ref, nd_ref):
    phiw = phiw_ref[...]
    wc = wc_ref[...]
    bct = bct_ref[...]
    cw = phiw * phic_ref[...]
    logits_t = lax.dot_general(
        wc, cw, (((1,), (1,)), ((), ())),
        preferred_element_type=jnp.float32) + bct
    yt = jax.nn.softmax(logits_t + gt_ref[...], axis=0)
    rows = lax.broadcasted_iota(jnp.int32, (_CATS, _E), 0)
    ymax = jnp.max(yt, axis=0, keepdims=True)
    cand = jnp.where(yt >= ymax, rows, jnp.int32(2**30))
    first = jnp.min(cand, axis=0, keepdims=True)
    zt = (rows == first).astype(jnp.float32)
    zt_ref[...] = zt
    pl_t = lax.dot_general(
        wc, phiw, (((1,), (1,)), ((), ())),
        preferred_element_type=jnp.float32) + bct
    priort_ref[...] = jax.nn.softmax(pl_t, axis=0)
    ndv = lax.dot_general(
        zt, wc, (((0,), (0,)), ((), ())),
        preferred_element_type=jnp.float32)
    nd_ref[...] = jnp.concatenate(
        [ndv, jnp.ones((_E, 1), jnp.float32)], axis=1)


_NBUF = 2
_KD = 4
_DNS = pl.cdiv(_SIZE, _BV)
_SUB = _BV // _KD
_EDGE = _SIZE - (_DNS - 1) * _BV
_EDGE_A = 1024
_EDGE_B = _EDGE - _EDGE_A


def _out_copies(out_buf, out_hbm, osems, buf, step, edge):
    if not edge:
        return tuple(
            pltpu.make_async_copy(
                out_buf.at[buf, pl.ds(k * _SUB, _SUB)],
                out_hbm.at[pl.ds(step * _BV + k * _SUB, _SUB)],
                osems.at[buf, k])
            for k in range(_KD))
    return (
        pltpu.make_async_copy(
            out_buf.at[buf, pl.ds(0, _EDGE_A)],
            out_hbm.at[pl.ds(step * _BV, _EDGE_A)],
            osems.at[buf, 0]),
        pltpu.make_async_copy(
            out_buf.at[buf, pl.ds(_EDGE_A, _EDGE_B)],
            out_hbm.at[pl.ds(step * _BV + _EDGE_A, _EDGE_B)],
            osems.at[buf, 1]),
    )


def _dec_body(wdt_ref, bdr_ref, nd_ref, out_hbm, out_buf, osems):
    st = pl.program_id(0)
    buf = lax.rem(st, _NBUF)

    @pl.when(jnp.logical_and(st >= _NBUF, st - _NBUF < _DNS - 1))
    def _wait_full():
        for cp in _out_copies(out_buf, out_hbm, osems, buf,
                              st - _NBUF, False):
            cp.wait()

    lhs = jnp.concatenate([wdt_ref[...], bdr_ref[...]], axis=0)
    out_buf[buf] = lax.dot_general(
        lhs, nd_ref[...], (((0,), (1,)), ((), ())),
        preferred_element_type=jnp.float32)

    @pl.when(st < _DNS - 1)
    def _start_full():
        for cp in _out_copies(out_buf, out_hbm, osems, buf, st, False):
            cp.start()

    @pl.when(st == _DNS - 1)
    def _edge_and_drain():
        for cp in _out_copies(out_buf, out_hbm, osems, buf, st, True):
            cp.start()
        prev = _DNS - 2
        for cp in _out_copies(out_buf, out_hbm, osems, prev % _NBUF,
                              prev, False):
            cp.wait()
        for cp in _out_copies(out_buf, out_hbm, osems, buf, st, True):
            cp.wait()


def kernel(w, c, edge_index, node_emb, W_comm, b_comm, W_dec, b_dec):
    del edge_index
    idx_all = jnp.concatenate([w, c]).astype(jnp.int32)
    phi = _sc_gather(node_emb, idx_all, 2 * _E, _DIM)
    phi_w, phi_c = phi[:_E], phi[_E:]

    gt = jax.random.gumbel(jax.random.key(42), (_E, _CATS), jnp.float32).T
    bct = b_comm.reshape(_CATS, 1)
    prior_t, z_t, nd = pl.pallas_call(
        _head_body,
        out_shape=(
            jax.ShapeDtypeStruct((_CATS, _E), jnp.float32),
            jax.ShapeDtypeStruct((_CATS, _E), jnp.float32),
            jax.ShapeDtypeStruct((_E, _DIM + 1), jnp.float32),
        ),
    )(phi_w, phi_c, W_comm, bct, gt)

    bdr = b_dec.reshape(1, _SIZE)
    recon_t = pl.pallas_call(
        _dec_body,
        grid=(_DNS,),
        in_specs=[
            pl.BlockSpec((_DIM, _BV), lambda i: (0, i)),
            pl.BlockSpec((1, _BV), lambda i: (0, i)),
            pl.BlockSpec((_E, _DIM + 1), lambda i: (0, 0)),
        ],
        out_specs=pl.BlockSpec(memory_space=pl.ANY),
        out_shape=jax.ShapeDtypeStruct((_SIZE, _E), jnp.float32),
        scratch_shapes=[
            pltpu.VMEM((_NBUF, _BV, _E), jnp.float32),
            pltpu.SemaphoreType.DMA((_NBUF, _KD)),
        ],
    )(W_dec.T, bdr, nd)

    return (prior_t.T, recon_t.T, z_t.T)


# speedup vs baseline: 1.0082x; 1.0005x over previous
import functools

import jax
import jax.numpy as jnp
from jax import lax
from jax.experimental import pallas as pl
from jax.experimental.pallas import tpu as pltpu
from jax.experimental.pallas import tpu_sc as plsc

_SIZE = 100000
_CATS = 100
_DIM = 64
_E = 1024

_BV = 6144


@functools.partial(jax.jit, static_argnums=(2, 3))
def _sc_gather(table, idx, B, D):
    info = plsc.get_sparse_core_info()
    NW = info.num_cores * info.num_subcores
    b_per_w = B // NW
    mesh = plsc.VectorSubcoreMesh(core_axis_name="c", subcore_axis_name="s")

    @functools.partial(
        pl.kernel,
        mesh=mesh,
        out_type=jax.ShapeDtypeStruct((B, D), jnp.float32),
        scratch_types=[
            pltpu.VMEM((b_per_w,), jnp.int32),
            pltpu.VMEM((b_per_w, D), jnp.float32),
            pltpu.SemaphoreType.DMA,
        ],
        compiler_params=pltpu.CompilerParams(use_tc_tiling_on_sc=False),
    )
    def k(table_hbm, idx_hbm, out_hbm, idx_v, rows_v, sem):
        wid = lax.axis_index("s") * info.num_cores + lax.axis_index("c")
        base = wid * b_per_w
        pltpu.sync_copy(idx_hbm.at[pl.ds(base, b_per_w)], idx_v)
        pltpu.async_copy(table_hbm.at[idx_v], rows_v, sem).wait()
        pltpu.sync_copy(rows_v, out_hbm.at[pl.ds(base, b_per_w)])

    return k(table, idx)


def _head_body(phiw_ref, phic_ref, wc_ref, bct_ref, gt_ref,
               priort_ref, zt_ref, nd_ref):
    phiw = phiw_ref[...]
    wc = wc_ref[...]
    bct = bct_ref[...]
    cw = phiw * phic_ref[...]
    logits_t = lax.dot_general(
        wc, cw, (((1,), (1,)), ((), ())),
        preferred_element_type=jnp.float32) + bct
    yt = jax.nn.softmax(logits_t + gt_ref[...], axis=0)
    rows = lax.broadcasted_iota(jnp.int32, (_CATS, _E), 0)
    ymax = jnp.max(yt, axis=0, keepdims=True)
    cand = jnp.where(yt >= ymax, rows, jnp.int32(2**30))
    first = jnp.min(cand, axis=0, keepdims=True)
    zt = (rows == first).astype(jnp.float32)
    zt_ref[...] = zt
    pl_t = lax.dot_general(
        wc, phiw, (((1,), (1,)), ((), ())),
        preferred_element_type=jnp.float32) + bct
    priort_ref[...] = jax.nn.softmax(pl_t, axis=0)
    ndv = lax.dot_general(
        zt, wc, (((0,), (0,)), ((), ())),
        preferred_element_type=jnp.float32)
    nd_ref[...] = jnp.concatenate(
        [ndv, jnp.ones((_E, 1), jnp.float32)], axis=1)


def _dec_body(wdt_ref, bdr_ref, nd_ref, out_ref):
    lhs = jnp.concatenate([wdt_ref[...], bdr_ref[...]], axis=0)
    out_ref[...] = lax.dot_general(
        lhs, nd_ref[...], (((0,), (1,)), ((), ())),
        preferred_element_type=jnp.float32)


def kernel(w, c, edge_index, node_emb, W_comm, b_comm, W_dec, b_dec):
    del edge_index
    idx_all = jnp.concatenate([w, c]).astype(jnp.int32)
    phi = _sc_gather(node_emb, idx_all, 2 * _E, _DIM)
    phi_w, phi_c = phi[:_E], phi[_E:]

    gt = jax.random.gumbel(jax.random.key(42), (_E, _CATS), jnp.float32).T
    bct = b_comm.reshape(_CATS, 1)
    prior_t, z_t, nd = pl.pallas_call(
        _head_body,
        out_shape=(
            jax.ShapeDtypeStruct((_CATS, _E), jnp.float32),
            jax.ShapeDtypeStruct((_CATS, _E), jnp.float32),
            jax.ShapeDtypeStruct((_E, _DIM + 1), jnp.float32),
        ),
    )(phi_w, phi_c, W_comm, bct, gt)

    bdr = b_dec.reshape(1, _SIZE)
    nb = pl.cdiv(_SIZE, _BV)
    recon_t = pl.pallas_call(
        _dec_body,
        grid=(nb,),
        in_specs=[
            pl.BlockSpec((_DIM, _BV), lambda i: (0, i)),
            pl.BlockSpec((1, _BV), lambda i: (0, i)),
            pl.BlockSpec((_E, _DIM + 1), lambda i: (0, 0)),
        ],
        out_specs=pl.BlockSpec((_BV, _E), lambda i: (i, 0)),
        out_shape=jax.ShapeDtypeStruct((_SIZE, _E), jnp.float32),
        compiler_params=pltpu.CompilerParams(
            dimension_semantics=("parallel",)),
    )(W_dec.T, bdr, nd)

    return (prior_t.T, recon_t.T, z_t.T)


# speedup vs baseline: 1.0082x; 1.0000x over previous
import functools

import jax
import jax.numpy as jnp
from jax import lax
from jax.experimental import pallas as pl
from jax.experimental.pallas import tpu as pltpu
from jax.experimental.pallas import tpu_sc as plsc

_SIZE = 100000
_CATS = 100
_DIM = 64
_E = 1024

_BV = 5120


@functools.partial(jax.jit, static_argnums=(2, 3))
def _sc_gather(table, idx, B, D):
    info = plsc.get_sparse_core_info()
    NW = info.num_cores * info.num_subcores
    b_per_w = B // NW
    mesh = plsc.VectorSubcoreMesh(core_axis_name="c", subcore_axis_name="s")

    @functools.partial(
        pl.kernel,
        mesh=mesh,
        out_type=jax.ShapeDtypeStruct((B, D), jnp.float32),
        scratch_types=[
            pltpu.VMEM((b_per_w,), jnp.int32),
            pltpu.VMEM((b_per_w, D), jnp.float32),
            pltpu.SemaphoreType.DMA,
        ],
        compiler_params=pltpu.CompilerParams(use_tc_tiling_on_sc=False),
    )
    def k(table_hbm, idx_hbm, out_hbm, idx_v, rows_v, sem):
        wid = lax.axis_index("s") * info.num_cores + lax.axis_index("c")
        base = wid * b_per_w
        pltpu.sync_copy(idx_hbm.at[pl.ds(base, b_per_w)], idx_v)
        pltpu.async_copy(table_hbm.at[idx_v], rows_v, sem).wait()
        pltpu.sync_copy(rows_v, out_hbm.at[pl.ds(base, b_per_w)])

    return k(table, idx)


def _head_body(phiw_ref, phic_ref, wc_ref, bct_ref, gt_ref,
               priort_ref, zt_ref, nd_ref):
    phiw = phiw_ref[...]
    wc = wc_ref[...]
    bct = bct_ref[...]
    cw = phiw * phic_ref[...]
    logits_t = lax.dot_general(
        wc, cw, (((1,), (1,)), ((), ())),
        preferred_element_type=jnp.float32) + bct
    yt = jax.nn.softmax(logits_t + gt_ref[...], axis=0)
    rows = lax.broadcasted_iota(jnp.int32, (_CATS, _E), 0)
    ymax = jnp.max(yt, axis=0, keepdims=True)
    cand = jnp.where(yt >= ymax, rows, jnp.int32(2**30))
    first = jnp.min(cand, axis=0, keepdims=True)
    zt = (rows == first).astype(jnp.float32)
    zt_ref[...] = zt
    pl_t = lax.dot_general(
        wc, phiw, (((1,), (1,)), ((), ())),
        preferred_element_type=jnp.float32) + bct
    priort_ref[...] = jax.nn.softmax(pl_t, axis=0)
    ndv = lax.dot_general(
        zt, wc, (((0,), (0,)), ((), ())),
        preferred_element_type=jnp.float32)
    nd_ref[...] = jnp.concatenate(
        [ndv, jnp.ones((_E, 1), jnp.float32)], axis=1)


def _dec_body(wdt_ref, bdr_ref, nd_ref, out_ref):
    lhs = jnp.concatenate([wdt_ref[...], bdr_ref[...]], axis=0)
    out_ref[...] = lax.dot_general(
        lhs, nd_ref[...], (((0,), (1,)), ((), ())),
        preferred_element_type=jnp.float32)


def kernel(w, c, edge_index, node_emb, W_comm, b_comm, W_dec, b_dec):
    del edge_index
    idx_all = jnp.concatenate([w, c]).astype(jnp.int32)
    phi = _sc_gather(node_emb, idx_all, 2 * _E, _DIM)
    phi_w, phi_c = phi[:_E], phi[_E:]

    gt = jax.random.gumbel(jax.random.key(42), (_E, _CATS), jnp.float32).T
    bct = b_comm.reshape(_CATS, 1)
    prior_t, z_t, nd = pl.pallas_call(
        _head_body,
        out_shape=(
            jax.ShapeDtypeStruct((_CATS, _E), jnp.float32),
            jax.ShapeDtypeStruct((_CATS, _E), jnp.float32),
            jax.ShapeDtypeStruct((_E, _DIM + 1), jnp.float32),
        ),
    )(phi_w, phi_c, W_comm, bct, gt)

    bdr = b_dec.reshape(1, _SIZE)
    nb = pl.cdiv(_SIZE, _BV)
    recon_t = pl.pallas_call(
        _dec_body,
        grid=(nb,),
        in_specs=[
            pl.BlockSpec((_DIM, _BV), lambda i: (0, i)),
            pl.BlockSpec((1, _BV), lambda i: (0, i)),
            pl.BlockSpec((_E, _DIM + 1), lambda i: (0, 0)),
        ],
        out_specs=pl.BlockSpec((_BV, _E), lambda i: (i, 0)),
        out_shape=jax.ShapeDtypeStruct((_SIZE, _E), jnp.float32),
        compiler_params=pltpu.CompilerParams(
            dimension_semantics=("parallel",)),
    )(W_dec.T, bdr, nd)

    return (prior_t.T, recon_t.T, z_t.T)
